# baseline (speedup 1.0000x reference)
import functools
import math

import numpy as np
import jax
from jax import lax
import jax.numpy as jnp
from jax.experimental import pallas as pl
from jax.experimental.pallas import tpu as pltpu
from jax.experimental.pallas import tpu_sc as plsc

_NT = 2048
_D = 2048
_NE = 16
_CAP = 256
_EPS = float(jnp.finfo(jnp.float32).eps)

_TB = 256
_NB = _NT // _TB

_ROWS = 2 * _NT
_NWORKERS = 32
_RPW = _ROWS // _NWORKERS
_NC = 2
_VEC = 16
_OUT_ROWS = _NT * _NE
_ZPW = _OUT_ROWS // _NWORKERS
_ZB = 256


def _gumbel_const():
    return jax.random.gumbel(jax.random.key(42), (_NT, _NE), dtype=jnp.float32)


def _cumsum_lanes(m):
    s = 1
    while s < _NT:
        m = m + jnp.pad(m[:, :-s], ((0, 0), (s, 0)))
        s *= 2
    return m


def _first_argmax_rows(vals, e_iota):
    vmax = jnp.max(vals, axis=0, keepdims=True)
    return jnp.min(jnp.where(vals == vmax, e_iota, _NE), axis=0, keepdims=True)


def _matmul_kernel(x_ref, wg_ref, out_ref):
    out_ref[...] = jnp.dot(x_ref[...], wg_ref[...],
                           preferred_element_type=jnp.float32)


def _route_kernel(lg_ref, gum_ref,
                  ridx_ref, lr_ref, wr_ref, vrows_ref, laux_ref):
    if True:
        logits = lg_ref[...].T
        lmax = jnp.max(logits, axis=0, keepdims=True)
        unnorm = jnp.exp(logits - lmax)
        gates = unnorm / jnp.sum(unnorm, axis=0, keepdims=True)

        e_iota = jax.lax.broadcasted_iota(jnp.int32, (_NE, _NT), 0)

        i1 = _first_argmax_rows(gates, e_iota)
        m1 = e_iota == i1
        mask1 = m1.astype(jnp.float32)

        noised = jnp.where(m1, -jnp.inf, logits + gum_ref[...])
        i2 = _first_argmax_rows(noised, e_iota)
        m2 = e_iota == i2
        mask2 = m2.astype(jnp.float32)

        cs1 = _cumsum_lanes(mask1)
        locations1 = cs1 - 1.0
        count1 = cs1[:, _NT - 1:_NT]
        locations2 = (_cumsum_lanes(mask2) - 1.0) + count1

        me = jnp.mean(gates, axis=1)
        ce = jnp.mean(mask1, axis=1)
        laux_ref[...] = (jnp.mean(me * ce) * (_NE * _NE)).reshape(1, 1)

        mask1 = mask1 * (locations1 < _CAP).astype(jnp.float32)
        mask2 = mask2 * (locations2 < _CAP).astype(jnp.float32)

        g1s = jnp.sum(gates * mask1, axis=0, keepdims=True)
        g2s = jnp.sum(gates * mask2, axis=0, keepdims=True)
        denom = jnp.maximum(g1s + g2s, _EPS)
        g1s = g1s / denom
        g2s = g2s / denom

        l1s = jnp.sum(locations1 * mask1, axis=0, keepdims=True).astype(jnp.int32)
        l2s = jnp.sum(locations2 * mask2, axis=0, keepdims=True).astype(jnp.int32)

        t_iota = jax.lax.broadcasted_iota(jnp.int32, (1, _NT), 1)
        r1 = t_iota * _NE + i1
        r2 = t_iota * _NE + i2

        ridx_ref[...] = jnp.concatenate([r1, r2], axis=0)

        lr_t = jnp.where(m1, l1s, l2s)
        wr_t = g1s * mask1 + g2s * mask2
        lr_ref[...] = lr_t.T
        wr_ref[...] = wr_t.T

        vl = jnp.concatenate([g1s, g2s], axis=0).T
        ll = jnp.concatenate([l1s, l2s], axis=0).T
        c_iota = jax.lax.broadcasted_iota(jnp.int32, (_NT, _CAP), 1)
        vrows_ref[0:_NT, :] = jnp.where(c_iota == ll[:, 0:1], vl[:, 0:1], 0.0)
        vrows_ref[_NT:_ROWS, :] = jnp.where(c_iota == ll[:, 1:2], vl[:, 1:2], 0.0)


def _zero_body(zrow_hbm, cw_hbm, zbuf_v, sem):
    wid = lax.axis_index("s") * _NC + lax.axis_index("c")
    base = wid * _ZPW
    pltpu.sync_copy(zrow_hbm, zbuf_v)
    copies = [
        pltpu.async_copy(zbuf_v, cw_hbm.at[pl.ds(base + k * _ZB, _ZB)], sem)
        for k in range(_ZPW // _ZB)
    ]
    for c in copies:
        c.wait()


def _scatter_body(vrows_hbm, ridx_hbm, cw_hbm, idx_v, vbuf_v, sem):
    wid = lax.axis_index("s") * _NC + lax.axis_index("c")
    base = wid * _RPW
    c1 = pltpu.async_copy(ridx_hbm.at[wid], idx_v, sem)
    c2 = pltpu.async_copy(vrows_hbm.at[pl.ds(base, _RPW)], vbuf_v, sem)
    c1.wait()
    c2.wait()
    pltpu.sync_copy(vbuf_v, cw_hbm.at[idx_v])


def kernel(x, Wg):
    gum_t = _gumbel_const().T
    mesh = plsc.VectorSubcoreMesh(core_axis_name="c", subcore_axis_name="s")

    zrow = jnp.zeros((_ZB, _CAP), jnp.float32)
    cw0 = pl.kernel(
        _zero_body,
        out_type=jax.ShapeDtypeStruct((_OUT_ROWS, _CAP), jnp.float32),
        mesh=mesh,
        scratch_types=[
            pltpu.VMEM((_ZB, _CAP), jnp.float32),
            pltpu.SemaphoreType.DMA,
        ],
    )(zrow)

    logits = pl.pallas_call(
        _matmul_kernel,
        grid=(_NB,),
        in_specs=[
            pl.BlockSpec((_TB, _D), lambda i: (i, 0)),
            pl.BlockSpec((_D, _NE), lambda i: (0, 0)),
        ],
        out_specs=pl.BlockSpec((_TB, _NE), lambda i: (i, 0)),
        out_shape=jax.ShapeDtypeStruct((_NT, _NE), jnp.float32),
    )(x, Wg)

    ridx, lr, wr, vrows, laux = pl.pallas_call(
        _route_kernel,
        out_shape=[
            jax.ShapeDtypeStruct((2, _NT), jnp.int32),
            jax.ShapeDtypeStruct((_NT, _NE), jnp.int32),
            jax.ShapeDtypeStruct((_NT, _NE), jnp.float32),
            jax.ShapeDtypeStruct((_ROWS, _CAP), jnp.float32),
            jax.ShapeDtypeStruct((1, 1), jnp.float32),
        ],
    )(logits, gum_t)

    ridx_tiled = ridx.reshape(_NWORKERS, _RPW)

    cw_ref = jax.new_ref(cw0)
    scatter = pl.kernel(
        _scatter_body,
        out_type=(),
        mesh=mesh,
        scratch_types=[
            pltpu.VMEM((_RPW,), jnp.int32),
            pltpu.VMEM((_RPW, _CAP), jnp.float32),
            pltpu.SemaphoreType.DMA,
        ],
    )
    scatter(vrows, ridx_tiled, cw_ref)

    cw = cw_ref[...].reshape(_NT, _NE, _CAP)

    c_iota = jax.lax.broadcasted_iota(jnp.int32, (_NT, _NE, _CAP), 2)
    dm = (c_iota == lr[:, :, None]) & (wr[:, :, None] > 0.0)
    return laux[0, 0], cw, dm


# speedup vs baseline: 1.0097x; 1.0097x over previous
import functools
import math

import numpy as np
import jax
from jax import lax
import jax.numpy as jnp
from jax.experimental import pallas as pl
from jax.experimental.pallas import tpu as pltpu
from jax.experimental.pallas import tpu_sc as plsc

_NT = 2048
_D = 2048
_NE = 16
_CAP = 256
_EPS = float(jnp.finfo(jnp.float32).eps)

_TB = 256
_NB = _NT // _TB

_ROWS = 2 * _NT
_NWORKERS = 32
_RPW = _ROWS // _NWORKERS
_NC = 2
_VEC = 16
_OUT_ROWS = _NT * _NE
_ZPW = _OUT_ROWS // _NWORKERS
_ZB = 256


def _gumbel_const():
    return jax.random.gumbel(jax.random.key(42), (_NT, _NE), dtype=jnp.float32)


def _cumsum_lanes(m):
    s = 1
    while s < _NT:
        m = m + jnp.pad(m[:, :-s], ((0, 0), (s, 0)))
        s *= 2
    return m


def _first_argmax_rows(vals, e_iota):
    vmax = jnp.max(vals, axis=0, keepdims=True)
    return jnp.min(jnp.where(vals == vmax, e_iota, _NE), axis=0, keepdims=True)


def _matmul_kernel(x0_ref, x1_ref, x2_ref, x3_ref, wg_ref, out_ref):
    q = _D // 4
    w = wg_ref[...]
    acc = jnp.dot(x0_ref[...], w[0:q], preferred_element_type=jnp.float32)
    acc += jnp.dot(x1_ref[...], w[q:2 * q], preferred_element_type=jnp.float32)
    acc += jnp.dot(x2_ref[...], w[2 * q:3 * q], preferred_element_type=jnp.float32)
    acc += jnp.dot(x3_ref[...], w[3 * q:4 * q], preferred_element_type=jnp.float32)
    out_ref[...] = acc


def _route_kernel(lg_ref, gum_ref,
                  ridx_ref, lr_ref, wr_ref, vrows_ref, laux_ref):
    if True:
        logits = lg_ref[...].T
        lmax = jnp.max(logits, axis=0, keepdims=True)
        unnorm = jnp.exp(logits - lmax)
        gates = unnorm / jnp.sum(unnorm, axis=0, keepdims=True)

        e_iota = jax.lax.broadcasted_iota(jnp.int32, (_NE, _NT), 0)

        i1 = _first_argmax_rows(gates, e_iota)
        m1 = e_iota == i1
        mask1 = m1.astype(jnp.float32)

        noised = jnp.where(m1, -jnp.inf, logits + gum_ref[...])
        i2 = _first_argmax_rows(noised, e_iota)
        m2 = e_iota == i2
        mask2 = m2.astype(jnp.float32)

        cs1 = _cumsum_lanes(mask1)
        locations1 = cs1 - 1.0
        count1 = cs1[:, _NT - 1:_NT]
        locations2 = (_cumsum_lanes(mask2) - 1.0) + count1

        me = jnp.mean(gates, axis=1)
        ce = jnp.mean(mask1, axis=1)
        laux_ref[...] = (jnp.mean(me * ce) * (_NE * _NE)).reshape(1, 1)

        mask1 = mask1 * (locations1 < _CAP).astype(jnp.float32)
        mask2 = mask2 * (locations2 < _CAP).astype(jnp.float32)

        g1s = jnp.sum(gates * mask1, axis=0, keepdims=True)
        g2s = jnp.sum(gates * mask2, axis=0, keepdims=True)
        denom = jnp.maximum(g1s + g2s, _EPS)
        g1s = g1s / denom
        g2s = g2s / denom

        l1s = jnp.sum(locations1 * mask1, axis=0, keepdims=True).astype(jnp.int32)
        l2s = jnp.sum(locations2 * mask2, axis=0, keepdims=True).astype(jnp.int32)

        t_iota = jax.lax.broadcasted_iota(jnp.int32, (1, _NT), 1)
        r1 = t_iota * _NE + i1
        r2 = t_iota * _NE + i2

        ridx_ref[...] = jnp.concatenate([r1, r2], axis=0)

        lr_t = jnp.where(m1, l1s, l2s)
        wr_t = g1s * mask1 + g2s * mask2
        lr_ref[...] = lr_t.T
        wr_ref[...] = wr_t.T

        vl = jnp.concatenate([g1s, g2s], axis=0).T
        ll = jnp.concatenate([l1s, l2s], axis=0).T
        c_iota = jax.lax.broadcasted_iota(jnp.int32, (_NT, _CAP), 1)
        vrows_ref[0:_NT, :] = jnp.where(c_iota == ll[:, 0:1], vl[:, 0:1], 0.0)
        vrows_ref[_NT:_ROWS, :] = jnp.where(c_iota == ll[:, 1:2], vl[:, 1:2], 0.0)


def _zero_body(zrow_hbm, cw_hbm, zbuf_v, sem):
    wid = lax.axis_index("s") * _NC + lax.axis_index("c")
    base = wid * _ZPW
    pltpu.sync_copy(zrow_hbm, zbuf_v)
    copies = [
        pltpu.async_copy(zbuf_v, cw_hbm.at[pl.ds(base + k * _ZB, _ZB)], sem)
        for k in range(_ZPW // _ZB)
    ]
    for c in copies:
        c.wait()


def _scatter_body(vrows_hbm, ridx_hbm, cw_hbm, idx_v, vbuf_v, sem):
    wid = lax.axis_index("s") * _NC + lax.axis_index("c")
    base = wid * _RPW
    c1 = pltpu.async_copy(ridx_hbm.at[wid], idx_v, sem)
    c2 = pltpu.async_copy(vrows_hbm.at[pl.ds(base, _RPW)], vbuf_v, sem)
    c1.wait()
    c2.wait()
    pltpu.sync_copy(vbuf_v, cw_hbm.at[idx_v])


def kernel(x, Wg):
    gum_t = _gumbel_const().T
    mesh = plsc.VectorSubcoreMesh(core_axis_name="c", subcore_axis_name="s")

    zrow = jnp.zeros((_ZB, _CAP), jnp.float32)
    cw0 = pl.kernel(
        _zero_body,
        out_type=jax.ShapeDtypeStruct((_OUT_ROWS, _CAP), jnp.float32),
        mesh=mesh,
        scratch_types=[
            pltpu.VMEM((_ZB, _CAP), jnp.float32),
            pltpu.SemaphoreType.DMA,
        ],
    )(zrow)

    q = _D // 4
    logits = pl.pallas_call(
        _matmul_kernel,
        grid=(_NB,),
        in_specs=[
            pl.BlockSpec((_TB, q), lambda i: (i, 0)),
            pl.BlockSpec((_TB, q), lambda i: (i, 1)),
            pl.BlockSpec((_TB, q), lambda i: (i, 2)),
            pl.BlockSpec((_TB, q), lambda i: (i, 3)),
            pl.BlockSpec((_D, _NE), lambda i: (0, 0)),
        ],
        out_specs=pl.BlockSpec((_TB, _NE), lambda i: (i, 0)),
        out_shape=jax.ShapeDtypeStruct((_NT, _NE), jnp.float32),
    )(x, x, x, x, Wg)

    ridx, lr, wr, vrows, laux = pl.pallas_call(
        _route_kernel,
        out_shape=[
            jax.ShapeDtypeStruct((2, _NT), jnp.int32),
            jax.ShapeDtypeStruct((_NT, _NE), jnp.int32),
            jax.ShapeDtypeStruct((_NT, _NE), jnp.float32),
            jax.ShapeDtypeStruct((_ROWS, _CAP), jnp.float32),
            jax.ShapeDtypeStruct((1, 1), jnp.float32),
        ],
    )(logits, gum_t)

    ridx_tiled = ridx.reshape(_NWORKERS, _RPW)

    cw_ref = jax.new_ref(cw0)
    scatter = pl.kernel(
        _scatter_body,
        out_type=(),
        mesh=mesh,
        scratch_types=[
            pltpu.VMEM((_RPW,), jnp.int32),
            pltpu.VMEM((_RPW, _CAP), jnp.float32),
            pltpu.SemaphoreType.DMA,
        ],
    )
    scatter(vrows, ridx_tiled, cw_ref)

    cw = cw_ref[...].reshape(_NT, _NE, _CAP)

    c_iota = jax.lax.broadcasted_iota(jnp.int32, (_NT, _NE, _CAP), 2)
    dm = (c_iota == lr[:, :, None]) & (wr[:, :, None] > 0.0)
    return laux[0, 0], cw, dm


# speedup vs baseline: 1.3117x; 1.2991x over previous
import functools
import math

import numpy as np
import jax
from jax import lax
import jax.numpy as jnp
from jax.experimental import pallas as pl
from jax.experimental.pallas import tpu as pltpu

_NT = 2048
_D = 2048
_NE = 16
_CAP = 256
_EPS = float(jnp.finfo(jnp.float32).eps)

_TB = 256
_NB = _NT // _TB
_CB = 128
_NCB = _NT // _CB


def _gumbel_const():
    return jax.random.gumbel(jax.random.key(42), (_NT, _NE), dtype=jnp.float32)


def _cumsum_lanes(m):
    s = 1
    while s < _NT:
        m = m + jnp.pad(m[:, :-s], ((0, 0), (s, 0)))
        s *= 2
    return m


def _first_argmax_rows(vals, e_iota):
    vmax = jnp.max(vals, axis=0, keepdims=True)
    return jnp.min(jnp.where(vals == vmax, e_iota, _NE), axis=0, keepdims=True)


def _matmul_kernel(x0_ref, x1_ref, x2_ref, x3_ref, wg_ref, out_ref):
    q = _D // 4
    w = wg_ref[...]
    acc = jnp.dot(x0_ref[...], w[0:q], preferred_element_type=jnp.float32)
    acc += jnp.dot(x1_ref[...], w[q:2 * q], preferred_element_type=jnp.float32)
    acc += jnp.dot(x2_ref[...], w[2 * q:3 * q], preferred_element_type=jnp.float32)
    acc += jnp.dot(x3_ref[...], w[3 * q:4 * q], preferred_element_type=jnp.float32)
    out_ref[...] = acc


def _route_kernel(lg_ref, gum_ref, lr_ref, wr_ref, laux_ref):
    logits = lg_ref[...].T
    lmax = jnp.max(logits, axis=0, keepdims=True)
    unnorm = jnp.exp(logits - lmax)
    gates = unnorm / jnp.sum(unnorm, axis=0, keepdims=True)

    e_iota = jax.lax.broadcasted_iota(jnp.int32, (_NE, _NT), 0)

    i1 = _first_argmax_rows(gates, e_iota)
    m1 = e_iota == i1
    mask1 = m1.astype(jnp.float32)

    noised = jnp.where(m1, -jnp.inf, logits + gum_ref[...])
    i2 = _first_argmax_rows(noised, e_iota)
    m2 = e_iota == i2
    mask2 = m2.astype(jnp.float32)

    cs1 = _cumsum_lanes(mask1)
    locations1 = cs1 - 1.0
    count1 = cs1[:, _NT - 1:_NT]
    locations2 = (_cumsum_lanes(mask2) - 1.0) + count1

    me = jnp.mean(gates, axis=1)
    ce = jnp.mean(mask1, axis=1)
    laux_ref[...] = (jnp.mean(me * ce) * (_NE * _NE)).reshape(1, 1)

    mask1 = mask1 * (locations1 < _CAP).astype(jnp.float32)
    mask2 = mask2 * (locations2 < _CAP).astype(jnp.float32)

    g1s = jnp.sum(gates * mask1, axis=0, keepdims=True)
    g2s = jnp.sum(gates * mask2, axis=0, keepdims=True)
    denom = jnp.maximum(g1s + g2s, _EPS)
    g1s = g1s / denom
    g2s = g2s / denom

    l1s = jnp.sum(locations1 * mask1, axis=0, keepdims=True).astype(jnp.int32)
    l2s = jnp.sum(locations2 * mask2, axis=0, keepdims=True).astype(jnp.int32)

    lr_t = jnp.where(m1, l1s, l2s)
    wr_t = g1s * mask1 + g2s * mask2
    lr_ref[...] = lr_t.T
    wr_ref[...] = wr_t.T


def _combine_kernel(lr_ref, wr_ref, cw_ref):
    lr = lr_ref[...]
    wr = wr_ref[...]
    c_iota = jax.lax.broadcasted_iota(jnp.int32, (_CB, _NE, _CAP), 2)
    cw_ref[...] = jnp.where(c_iota == lr[:, :, None], wr[:, :, None], 0.0)


def kernel(x, Wg):
    gum_t = _gumbel_const().T

    q = _D // 4
    logits = pl.pallas_call(
        _matmul_kernel,
        grid=(_NB,),
        in_specs=[
            pl.BlockSpec((_TB, q), lambda i: (i, 0)),
            pl.BlockSpec((_TB, q), lambda i: (i, 1)),
            pl.BlockSpec((_TB, q), lambda i: (i, 2)),
            pl.BlockSpec((_TB, q), lambda i: (i, 3)),
            pl.BlockSpec((_D, _NE), lambda i: (0, 0)),
        ],
        out_specs=pl.BlockSpec((_TB, _NE), lambda i: (i, 0)),
        out_shape=jax.ShapeDtypeStruct((_NT, _NE), jnp.float32),
    )(x, x, x, x, Wg)

    lr, wr, laux = pl.pallas_call(
        _route_kernel,
        out_shape=[
            jax.ShapeDtypeStruct((_NT, _NE), jnp.int32),
            jax.ShapeDtypeStruct((_NT, _NE), jnp.float32),
            jax.ShapeDtypeStruct((1, 1), jnp.float32),
        ],
    )(logits, gum_t)

    tokb = pl.BlockSpec((_CB, _NE), lambda i: (i, 0))
    cw = pl.pallas_call(
        _combine_kernel,
        grid=(_NCB,),
        in_specs=[tokb, tokb],
        out_specs=pl.BlockSpec((_CB, _NE, _CAP), lambda i: (i, 0, 0)),
        out_shape=jax.ShapeDtypeStruct((_NT, _NE, _CAP), jnp.float32),
    )(lr, wr)

    c_iota = jax.lax.broadcasted_iota(jnp.int32, (_NT, _NE, _CAP), 2)
    dm = (c_iota == lr[:, :, None]) & (wr[:, :, None] > 0.0)
    return laux[0, 0], cw, dm


# speedup vs baseline: 1.3339x; 1.0169x over previous
import functools
import math

import numpy as np
import jax
from jax import lax
import jax.numpy as jnp
from jax.experimental import pallas as pl
from jax.experimental.pallas import tpu as pltpu

_NT = 2048
_D = 2048
_NE = 16
_CAP = 256
_EPS = float(jnp.finfo(jnp.float32).eps)

_TB = 256
_NB = _NT // _TB
_CB = 128
_NCB = _NT // _CB


def _gumbel_const():
    return jax.random.gumbel(jax.random.key(42), (_NT, _NE), dtype=jnp.float32)


def _cumsum_lanes(m):
    s = 1
    while s < _NT:
        m = m + jnp.pad(m[:, :-s], ((0, 0), (s, 0)))
        s *= 2
    return m


def _first_argmax_rows(vals, e_iota):
    vmax = jnp.max(vals, axis=0, keepdims=True)
    return jnp.min(jnp.where(vals == vmax, e_iota, _NE), axis=0, keepdims=True)


def _matmul_kernel(x0_ref, x1_ref, x2_ref, x3_ref, wg_ref, out_ref):
    q = _D // 4
    w = wg_ref[...]
    acc = jnp.dot(x0_ref[...], w[0:q], preferred_element_type=jnp.float32)
    acc += jnp.dot(x1_ref[...], w[q:2 * q], preferred_element_type=jnp.float32)
    acc += jnp.dot(x2_ref[...], w[2 * q:3 * q], preferred_element_type=jnp.float32)
    acc += jnp.dot(x3_ref[...], w[3 * q:4 * q], preferred_element_type=jnp.float32)
    out_ref[...] = acc


def _route_kernel(lg_ref, gum_ref, lr_ref, wr_ref, laux_ref):
    logits = lg_ref[...].T
    lmax = jnp.max(logits, axis=0, keepdims=True)
    unnorm = jnp.exp(logits - lmax)
    gates = unnorm / jnp.sum(unnorm, axis=0, keepdims=True)

    e_iota = jax.lax.broadcasted_iota(jnp.int32, (_NE, _NT), 0)

    i1 = _first_argmax_rows(gates, e_iota)
    m1 = e_iota == i1
    mask1 = m1.astype(jnp.float32)

    noised = jnp.where(m1, -jnp.inf, logits + gum_ref[...])
    i2 = _first_argmax_rows(noised, e_iota)
    m2 = e_iota == i2
    mask2 = m2.astype(jnp.float32)

    cs1 = _cumsum_lanes(mask1)
    locations1 = cs1 - 1.0
    count1 = cs1[:, _NT - 1:_NT]
    locations2 = (_cumsum_lanes(mask2) - 1.0) + count1

    me = jnp.mean(gates, axis=1)
    ce = jnp.mean(mask1, axis=1)
    laux_ref[...] = (jnp.mean(me * ce) * (_NE * _NE)).reshape(1, 1)

    mask1 = mask1 * (locations1 < _CAP).astype(jnp.float32)
    mask2 = mask2 * (locations2 < _CAP).astype(jnp.float32)

    g1s = jnp.sum(gates * mask1, axis=0, keepdims=True)
    g2s = jnp.sum(gates * mask2, axis=0, keepdims=True)
    denom = jnp.maximum(g1s + g2s, _EPS)
    g1s = g1s / denom
    g2s = g2s / denom

    l1s = jnp.sum(locations1 * mask1, axis=0, keepdims=True).astype(jnp.int32)
    l2s = jnp.sum(locations2 * mask2, axis=0, keepdims=True).astype(jnp.int32)

    lr_t = jnp.where(m1, l1s, l2s)
    wr_t = g1s * mask1 + g2s * mask2
    lr_ref[...] = lr_t.T
    wr_ref[...] = wr_t.T


def _combine_kernel(lr_ref, wr_ref, cw_ref, dm_ref):
    lr = lr_ref[...]
    wr = wr_ref[...]
    c_iota = jax.lax.broadcasted_iota(jnp.int32, (_CB, _NE, _CAP), 2)
    oneh = c_iota == lr[:, :, None]
    cw = jnp.where(oneh, wr[:, :, None], 0.0)
    cw_ref[...] = cw
    dm_ref[...] = (cw > 0.0).astype(jnp.int8)


def kernel(x, Wg):
    gum_t = _gumbel_const().T

    q = _D // 4
    logits = pl.pallas_call(
        _matmul_kernel,
        grid=(_NB,),
        in_specs=[
            pl.BlockSpec((_TB, q), lambda i: (i, 0)),
            pl.BlockSpec((_TB, q), lambda i: (i, 1)),
            pl.BlockSpec((_TB, q), lambda i: (i, 2)),
            pl.BlockSpec((_TB, q), lambda i: (i, 3)),
            pl.BlockSpec((_D, _NE), lambda i: (0, 0)),
        ],
        out_specs=pl.BlockSpec((_TB, _NE), lambda i: (i, 0)),
        out_shape=jax.ShapeDtypeStruct((_NT, _NE), jnp.float32),
    )(x, x, x, x, Wg)

    lr, wr, laux = pl.pallas_call(
        _route_kernel,
        out_shape=[
            jax.ShapeDtypeStruct((_NT, _NE), jnp.int32),
            jax.ShapeDtypeStruct((_NT, _NE), jnp.float32),
            jax.ShapeDtypeStruct((1, 1), jnp.float32),
        ],
    )(logits, gum_t)

    tokb = pl.BlockSpec((_CB, _NE), lambda i: (i, 0))
    big = pl.BlockSpec((_CB, _NE, _CAP), lambda i: (i, 0, 0))
    cw, dm8 = pl.pallas_call(
        _combine_kernel,
        grid=(_NCB,),
        in_specs=[tokb, tokb],
        out_specs=[big, big],
        out_shape=[
            jax.ShapeDtypeStruct((_NT, _NE, _CAP), jnp.float32),
            jax.ShapeDtypeStruct((_NT, _NE, _CAP), jnp.int8),
        ],
    )(lr, wr)

    dm = dm8.astype(jnp.bool_)
    return laux[0, 0], cw, dm


# speedup vs baseline: 1.4484x; 1.0858x over previous
import functools
import math

import numpy as np
import jax
from jax import lax
import jax.numpy as jnp
from jax.experimental import pallas as pl
from jax.experimental.pallas import tpu as pltpu

_NT = 2048
_D = 2048
_NE = 16
_CAP = 256
_EPS = float(jnp.finfo(jnp.float32).eps)

_TB = 256
_NB = _NT // _TB
_CB = 128
_NCB = _NT // _CB


def _gumbel_const():
    return jax.random.gumbel(jax.random.key(42), (_NT, _NE), dtype=jnp.float32)


def _cumsum_lanes(m):
    s = 1
    while s < _NT:
        m = m + jnp.pad(m[:, :-s], ((0, 0), (s, 0)))
        s *= 2
    return m


def _first_argmax_rows(vals, e_iota):
    vmax = jnp.max(vals, axis=0, keepdims=True)
    return jnp.min(jnp.where(vals == vmax, e_iota, _NE), axis=0, keepdims=True)


def _matmul_kernel(x0_ref, x1_ref, x2_ref, x3_ref, wg_ref, out_ref):
    q = _D // 4
    w = wg_ref[...]
    acc = jnp.dot(x0_ref[...], w[0:q], preferred_element_type=jnp.float32)
    acc += jnp.dot(x1_ref[...], w[q:2 * q], preferred_element_type=jnp.float32)
    acc += jnp.dot(x2_ref[...], w[2 * q:3 * q], preferred_element_type=jnp.float32)
    acc += jnp.dot(x3_ref[...], w[3 * q:4 * q], preferred_element_type=jnp.float32)
    out_ref[...] = acc


def _routing_tables(lg_ref, gum_ref, laux_ref, lr_scr, wr_scr):
    logits = lg_ref[...].T
    lmax = jnp.max(logits, axis=0, keepdims=True)
    unnorm = jnp.exp(logits - lmax)
    gates = unnorm / jnp.sum(unnorm, axis=0, keepdims=True)

    e_iota = jax.lax.broadcasted_iota(jnp.int32, (_NE, _NT), 0)

    i1 = _first_argmax_rows(gates, e_iota)
    m1 = e_iota == i1
    mask1 = m1.astype(jnp.float32)

    noised = jnp.where(m1, -jnp.inf, logits + gum_ref[...])
    i2 = _first_argmax_rows(noised, e_iota)
    m2 = e_iota == i2
    mask2 = m2.astype(jnp.float32)

    cs1 = _cumsum_lanes(mask1)
    locations1 = cs1 - 1.0
    count1 = cs1[:, _NT - 1:_NT]
    locations2 = (_cumsum_lanes(mask2) - 1.0) + count1

    me = jnp.mean(gates, axis=1)
    ce = jnp.mean(mask1, axis=1)
    laux_ref[...] = (jnp.mean(me * ce) * (_NE * _NE)).reshape(1, 1)

    mask1 = mask1 * (locations1 < _CAP).astype(jnp.float32)
    mask2 = mask2 * (locations2 < _CAP).astype(jnp.float32)

    g1s = jnp.sum(gates * mask1, axis=0, keepdims=True)
    g2s = jnp.sum(gates * mask2, axis=0, keepdims=True)
    denom = jnp.maximum(g1s + g2s, _EPS)
    g1s = g1s / denom
    g2s = g2s / denom

    l1s = jnp.sum(locations1 * mask1, axis=0, keepdims=True).astype(jnp.int32)
    l2s = jnp.sum(locations2 * mask2, axis=0, keepdims=True).astype(jnp.int32)

    lr_t = jnp.where(m1, l1s, l2s)
    wr_t = g1s * mask1 + g2s * mask2
    lr_scr[...] = lr_t.T
    wr_scr[...] = wr_t.T


def _combine_kernel(lg_ref, gum_ref, cw_ref, dm_ref, laux_ref, lr_scr, wr_scr):
    i = pl.program_id(0)

    @pl.when(i == 0)
    def _():
        _routing_tables(lg_ref, gum_ref, laux_ref, lr_scr, wr_scr)

    lr = lr_scr[pl.ds(i * _CB, _CB), :]
    wr = wr_scr[pl.ds(i * _CB, _CB), :]
    c_iota = jax.lax.broadcasted_iota(jnp.int32, (_CB, _NE, _CAP), 2)
    oneh = c_iota == lr[:, :, None]
    cw = jnp.where(oneh, wr[:, :, None], 0.0)
    cw_ref[...] = cw
    dm_ref[...] = (cw > 0.0).astype(jnp.int8)


def kernel(x, Wg):
    gum_t = _gumbel_const().T

    q = _D // 4
    logits = pl.pallas_call(
        _matmul_kernel,
        grid=(_NB,),
        in_specs=[
            pl.BlockSpec((_TB, q), lambda i: (i, 0)),
            pl.BlockSpec((_TB, q), lambda i: (i, 1)),
            pl.BlockSpec((_TB, q), lambda i: (i, 2)),
            pl.BlockSpec((_TB, q), lambda i: (i, 3)),
            pl.BlockSpec((_D, _NE), lambda i: (0, 0)),
        ],
        out_specs=pl.BlockSpec((_TB, _NE), lambda i: (i, 0)),
        out_shape=jax.ShapeDtypeStruct((_NT, _NE), jnp.float32),
    )(x, x, x, x, Wg)

    big = pl.BlockSpec((_CB, _NE, _CAP), lambda i: (i, 0, 0))
    cw, dm8, laux = pl.pallas_call(
        _combine_kernel,
        grid=(_NCB,),
        in_specs=[
            pl.BlockSpec((_NT, _NE), lambda i: (0, 0)),
            pl.BlockSpec((_NE, _NT), lambda i: (0, 0)),
        ],
        out_specs=[big, big, pl.BlockSpec((1, 1), lambda i: (0, 0))],
        out_shape=[
            jax.ShapeDtypeStruct((_NT, _NE, _CAP), jnp.float32),
            jax.ShapeDtypeStruct((_NT, _NE, _CAP), jnp.int8),
            jax.ShapeDtypeStruct((1, 1), jnp.float32),
        ],
        scratch_shapes=[
            pltpu.VMEM((_NT, _NE), jnp.int32),
            pltpu.VMEM((_NT, _NE), jnp.float32),
        ],
    )(logits, gum_t)

    dm = dm8.astype(jnp.bool_)
    return laux[0, 0], cw, dm
